# Initial kernel scaffold; baseline (speedup 1.0000x reference)
import jax
import jax.numpy as jnp
from jax.experimental import pallas as pl


def kernel(x, selected_index, support, w):
    raise NotImplementedError("write your pallas kernel here")


# speedup vs baseline: 3.7762x; 3.7762x over previous
import functools

import jax
import jax.numpy as jnp
from jax import lax
from jax.experimental import pallas as pl
from jax.experimental.pallas import tpu as pltpu
from jax.experimental.pallas import tpu_sc as plsc


_BM = 400


def _mm_body(s_ref, x_ref, w_ref, o_ref):
    sx = jnp.dot(s_ref[...], x_ref[...], preferred_element_type=jnp.float32)
    o_ref[...] = jnp.dot(sx, w_ref[...], preferred_element_type=jnp.float32)


def _spmm(sup, x, w):
    n, k = sup.shape
    d = w.shape[1]
    return pl.pallas_call(
        _mm_body,
        grid=(n // _BM,),
        in_specs=[
            pl.BlockSpec((_BM, k), lambda i: (i, 0)),
            pl.BlockSpec((k, x.shape[1]), lambda i: (0, 0)),
            pl.BlockSpec(w.shape, lambda i: (0, 0)),
        ],
        out_specs=pl.BlockSpec((_BM, d), lambda i: (i, 0)),
        out_shape=jax.ShapeDtypeStruct((n, d), jnp.float32),
    )(sup, x, w)


_NW = 32
_CHUNK = 80


def _make_gather(d, cpw):
    mesh = plsc.VectorSubcoreMesh(core_axis_name="c", subcore_axis_name="s")
    b_pad = _NW * _CHUNK * cpw

    @functools.partial(
        pl.kernel,
        mesh=mesh,
        out_type=jax.ShapeDtypeStruct((b_pad, d), jnp.float32),
        scratch_types=[
            pltpu.VMEM((cpw, _CHUNK), jnp.int32),
            pltpu.VMEM((cpw, _CHUNK, d), jnp.float32),
            pltpu.SemaphoreType.DMA,
        ],
    )
    def gk(table_hbm, idx_hbm, out_hbm, idx_v, rows_v, sem):
        wid = lax.axis_index("s") * 2 + lax.axis_index("c")
        base = wid * (cpw * _CHUNK)
        pltpu.sync_copy(idx_hbm.at[pl.ds(wid * cpw, cpw)], idx_v)
        copies = [
            pltpu.async_copy(table_hbm.at[idx_v.at[c]], rows_v.at[c], sem)
            for c in range(cpw)
        ]
        for c in range(cpw):
            copies[c].wait()
            pltpu.sync_copy(
                rows_v.at[c], out_hbm.at[pl.ds(base + c * _CHUNK, _CHUNK)]
            )

    return gk


def kernel(x, selected_index, support, w):
    n = x.shape[0]
    s = _spmm(support[0], x, w)
    for i in range(1, support.shape[0]):
        s = s + _spmm(support[i], x, w)

    per_w = _NW * _CHUNK
    cpw = -(-n // per_w)
    b_pad = per_w * cpw
    idx = jnp.zeros((b_pad,), jnp.int32).at[:n].set(
        selected_index.astype(jnp.int32))
    idx2d = idx.reshape(_NW * cpw, _CHUNK)
    out_pad = _make_gather(w.shape[1], cpw)(s, idx2d)
    return out_pad[:n]
